# baseline (speedup 1.0000x reference)
import jax
import jax.numpy as jnp
from jax.experimental import pallas as pl
from jax.experimental.pallas import tpu as pltpu

M_COLS = 28
MPAD = 32
ALPHA_SLOPE = 0.2
BN = 10000
PAR = 1


def _gat_kernel(x_ref, edge_ref, x28_ref, w_ref, a_ref,
                out_ref, h28_sc, w1_sc, s2_sc):
    j = pl.program_id(1)
    d = w_ref.shape[0]

    @pl.when(j == 0)
    def _prologue():
        w = w_ref[...]
        h28 = jax.lax.dot_general(x28_ref[...], w, (((1,), (1,)), ((), ())),
                                  preferred_element_type=jnp.float32)
        h28_sc[...] = h28
        w1_sc[...] = jnp.dot(a_ref[:, :d], w, preferred_element_type=jnp.float32)
        s2_sc[...] = jax.lax.dot_general(a_ref[:, d:], h28,
                                         (((1,), (1,)), ((), ())),
                                         preferred_element_type=jnp.float32)

    s1 = jnp.sum(x_ref[...] * w1_sc[...], axis=1, keepdims=True)
    logits = s1 + s2_sc[...]
    lrelu = jnp.where(logits >= 0, logits, ALPHA_SLOPE * logits)
    edge_p = jnp.concatenate(
        [edge_ref[...], jnp.zeros((edge_ref.shape[0], MPAD - M_COLS),
                                  jnp.int32)], axis=1)
    e = jnp.where(edge_p != 0, jnp.exp(-lrelu), 0.0)
    rowsum = jnp.sum(e, axis=1, keepdims=True)
    hp = jnp.dot(e, h28_sc[...], preferred_element_type=jnp.float32) / rowsum
    out_ref[...] = jnp.where(hp > 0, hp, jnp.exp(hp) - 1.0)


def kernel(x, edge, W, a):
    n, d_in = x.shape
    d_out = W.shape[0]
    steps = n // (BN * PAR)
    return pl.pallas_call(
        _gat_kernel,
        grid=(PAR, steps),
        in_specs=[
            pl.BlockSpec((BN, d_in), lambda i, j: (i * (n // (BN * PAR)) + j, 0)),
            pl.BlockSpec((BN, M_COLS), lambda i, j: (i * (n // (BN * PAR)) + j, 0)),
            pl.BlockSpec((MPAD, d_in), lambda i, j: (0, 0)),
            pl.BlockSpec((d_out, d_in), lambda i, j: (0, 0)),
            pl.BlockSpec((1, 2 * d_out), lambda i, j: (0, 0)),
        ],
        out_specs=pl.BlockSpec((BN, d_out), lambda i, j: (i * (n // (BN * PAR)) + j, 0)),
        out_shape=jax.ShapeDtypeStruct((n, d_out), jnp.float32),
        scratch_shapes=[
            pltpu.VMEM((MPAD, d_out), jnp.float32),
            pltpu.VMEM((1, d_out), jnp.float32),
            pltpu.VMEM((1, MPAD), jnp.float32),
        ],
        compiler_params=pltpu.CompilerParams(
            dimension_semantics=("parallel", "arbitrary")),
    )(x, edge, x, W, a)


# speedup vs baseline: 1.2658x; 1.2658x over previous
import jax
import jax.numpy as jnp
from jax.experimental import pallas as pl
from jax.experimental.pallas import tpu as pltpu

M_COLS = 28
MPAD = 32
ALPHA_SLOPE = 0.2
BN = 5000
PAR = 1


def _gat_kernel(x_ref, edge_ref, x28_ref, w_ref, a_ref,
                out_ref, h28_sc, w1_sc, s2_sc):
    j = pl.program_id(1)
    d = w_ref.shape[0]

    @pl.when(j == 0)
    def _prologue():
        w = w_ref[...]
        h28 = jax.lax.dot_general(x28_ref[...], w, (((1,), (1,)), ((), ())),
                                  preferred_element_type=jnp.float32)
        h28_sc[...] = h28
        w1_sc[...] = jnp.dot(a_ref[:, :d], w, preferred_element_type=jnp.float32)
        s2_sc[...] = jax.lax.dot_general(a_ref[:, d:], h28,
                                         (((1,), (1,)), ((), ())),
                                         preferred_element_type=jnp.float32)

    s1 = jnp.sum(x_ref[...] * w1_sc[...], axis=1, keepdims=True)
    logits = s1 + s2_sc[...]
    lrelu = jnp.where(logits >= 0, logits, ALPHA_SLOPE * logits)
    edge_p = jnp.concatenate(
        [edge_ref[...], jnp.zeros((edge_ref.shape[0], MPAD - M_COLS),
                                  jnp.int32)], axis=1)
    e = jnp.where(edge_p != 0, jnp.exp(-lrelu), 0.0)
    rowsum = jnp.sum(e, axis=1, keepdims=True)
    hp = jnp.dot(e, h28_sc[...], preferred_element_type=jnp.float32) / rowsum
    out_ref[...] = jnp.where(hp > 0, hp, jnp.exp(hp) - 1.0)


def kernel(x, edge, W, a):
    n, d_in = x.shape
    d_out = W.shape[0]
    steps = n // (BN * PAR)
    return pl.pallas_call(
        _gat_kernel,
        grid=(PAR, steps),
        in_specs=[
            pl.BlockSpec((BN, d_in), lambda i, j: (i * (n // (BN * PAR)) + j, 0)),
            pl.BlockSpec((BN, M_COLS), lambda i, j: (i * (n // (BN * PAR)) + j, 0)),
            pl.BlockSpec((MPAD, d_in), lambda i, j: (0, 0)),
            pl.BlockSpec((d_out, d_in), lambda i, j: (0, 0)),
            pl.BlockSpec((1, 2 * d_out), lambda i, j: (0, 0)),
        ],
        out_specs=pl.BlockSpec((BN, d_out), lambda i, j: (i * (n // (BN * PAR)) + j, 0)),
        out_shape=jax.ShapeDtypeStruct((n, d_out), jnp.float32),
        scratch_shapes=[
            pltpu.VMEM((MPAD, d_out), jnp.float32),
            pltpu.VMEM((1, d_out), jnp.float32),
            pltpu.VMEM((1, MPAD), jnp.float32),
        ],
        compiler_params=pltpu.CompilerParams(
            dimension_semantics=("parallel", "arbitrary")),
    )(x, edge, x, W, a)
